# baseline (speedup 1.0000x reference)
import functools

import jax
import jax.numpy as jnp
from jax import lax
from jax.experimental import pallas as pl
from jax.experimental.pallas import tpu as pltpu
from jax.experimental.pallas import tpu_sc as plsc

NC = 2
NS = 16
NW = NC * NS
CH = 128
K = 4
BLK = CH * K
PC = 8192
PH = PC // 2


def _pack_table_tc(tableT):
    D, V = tableT.shape
    NB = (V + PC - 1) // PC
    def body(x_ref, o_ref):
        x = x_ref[...]
        ya = x[:, :PH].T
        yb = x[:, PH:].T
        o_ref[...] = jnp.concatenate([ya, yb], axis=1)

    return pl.pallas_call(
        body,
        grid=(NB,),
        in_specs=[
            pl.BlockSpec((D, PC), lambda i: (0, i)),
        ],
        out_specs=pl.BlockSpec((PH, 2 * D), lambda i: (i, 0)),
        out_shape=jax.ShapeDtypeStruct((NB * PH, 2 * D), jnp.float32),
    )(tableT)


def _gather_sc(table, idx2d):
    VP, D = table.shape
    B = idx2d.shape[0] * CH
    b_per_w = B // NW
    n_blk = b_per_w // BLK
    rows_per_w = b_per_w // CH

    mesh = plsc.VectorSubcoreMesh(
        core_axis_name="c", subcore_axis_name="s", num_cores=NC, num_subcores=NS
    )

    @functools.partial(
        pl.kernel,
        out_type=jax.ShapeDtypeStruct((B, D), jnp.float32),
        mesh=mesh,
        compiler_params=pltpu.CompilerParams(use_tc_tiling_on_sc=False),
        scratch_types=[
            pltpu.VMEM((rows_per_w, CH), jnp.int32),
            pltpu.VMEM((BLK, D), jnp.float32),
            pltpu.SemaphoreType.DMA,
        ],
    )
    def grab(table_hbm, idx_hbm, out_hbm, idx_v, buf, sem):
        wid = lax.axis_index("s") * NC + lax.axis_index("c")
        base = wid * b_per_w
        pltpu.sync_copy(idx_hbm.at[pl.ds(wid * rows_per_w, rows_per_w)], idx_v)

        def remap(j, _):
            def remap16(c, _):
                r = idx_v[j, pl.ds(c * 16, 16)]
                p = lax.bitwise_and(r, PC - 1)
                gbase = lax.sub(r, p)
                off = lax.select(
                    p < PH, lax.shift_left(p, 1), 2 * p - (PC - 1)
                )
                idx_v[j, pl.ds(c * 16, 16)] = lax.add(gbase, off)
                return ()

            lax.fori_loop(0, CH // 16, remap16, (), unroll=True)
            return ()

        lax.fori_loop(0, rows_per_w, remap, (), unroll=False)

        def block(g, _):
            for b in range(K):
                pltpu.async_copy(
                    table_hbm.at[idx_v.at[g * K + b]],
                    buf.at[pl.ds(b * CH, CH)],
                    sem,
                )
            for b in range(K):
                pltpu.make_async_copy(
                    table_hbm.at[idx_v.at[g * K + b]],
                    buf.at[pl.ds(b * CH, CH)],
                    sem,
                ).wait()
            pltpu.sync_copy(buf, out_hbm.at[pl.ds(base + g * BLK, BLK)])
            return ()

        lax.fori_loop(0, n_blk, block, (), unroll=False)

    return grab(table, idx2d)


def _project_tc(e2, wbig, Bb, Ll):
    B2, D2 = e2.shape
    KV2 = wbig.shape[1]
    KV = KV2 // 2
    GB = 64
    rows2 = GB * Ll // 2

    def body(x_ref, w_ref, o_ref):
        y = lax.dot_general(
            x_ref[...], w_ref[...],
            dimension_numbers=(((1,), (0,)), ((), ())),
            preferred_element_type=jnp.float32,
        )
        o_ref[...] = y.reshape(rows2, 2, KV).reshape(2 * rows2, KV).reshape(
            GB, Ll, KV
        )

    return pl.pallas_call(
        body,
        grid=(Bb // GB,),
        in_specs=[
            pl.BlockSpec((rows2, D2), lambda i: (i, 0)),
            pl.BlockSpec((D2, KV2), lambda i: (0, 0)),
        ],
        out_specs=pl.BlockSpec((GB, Ll, KV), lambda i: (i, 0, 0)),
        out_shape=jax.ShapeDtypeStruct((Bb, Ll, KV), jnp.float32),
    )(e2, wbig)


def kernel(input_ids, embed_weight, proj_weight):
    Bb, Ll = input_ids.shape
    KV, D = proj_weight.shape
    idx2d = input_ids.reshape(-1, CH).astype(jnp.int32)
    tableP = _pack_table_tc(embed_weight.T)
    table_lin = tableP.reshape(tableP.shape[0] * 2, D)
    e = _gather_sc(table_lin, idx2d)
    e2 = e.reshape(e.shape[0] // 2, 2 * e.shape[1])
    wt = proj_weight.T
    zero = jnp.zeros((D, KV), jnp.float32)
    wbig = jnp.block([[wt, zero], [zero, wt]])
    return _project_tc(e2, wbig, Bb, Ll)


# speedup vs baseline: 1.0529x; 1.0529x over previous
import functools

import jax
import jax.numpy as jnp
from jax import lax
from jax.experimental import pallas as pl
from jax.experimental.pallas import tpu as pltpu
from jax.experimental.pallas import tpu_sc as plsc

NC = 2
NS = 16
NW = NC * NS
CH = 128
K = 4
BLK = CH * K
PC = 16384
PH = PC // 2


def _pack_table_tc(tableT):
    D, V = tableT.shape
    NB = (V + PC - 1) // PC
    def body(x_ref, o_ref):
        x = x_ref[...]
        ya = x[:, :PH].T
        yb = x[:, PH:].T
        o_ref[...] = jnp.concatenate([ya, yb], axis=1)

    return pl.pallas_call(
        body,
        grid=(NB,),
        in_specs=[
            pl.BlockSpec((D, PC), lambda i: (0, i)),
        ],
        out_specs=pl.BlockSpec((PH, 2 * D), lambda i: (i, 0)),
        out_shape=jax.ShapeDtypeStruct((NB * PH, 2 * D), jnp.float32),
    )(tableT)


def _gather_sc(table, idx2d):
    VP, D = table.shape
    B = idx2d.shape[0] * CH
    b_per_w = B // NW
    n_blk = b_per_w // BLK
    rows_per_w = b_per_w // CH

    mesh = plsc.VectorSubcoreMesh(
        core_axis_name="c", subcore_axis_name="s", num_cores=NC, num_subcores=NS
    )

    @functools.partial(
        pl.kernel,
        out_type=jax.ShapeDtypeStruct((B, D), jnp.float32),
        mesh=mesh,
        compiler_params=pltpu.CompilerParams(use_tc_tiling_on_sc=False),
        scratch_types=[
            pltpu.VMEM((rows_per_w, CH), jnp.int32),
            pltpu.VMEM((BLK, D), jnp.float32),
            pltpu.SemaphoreType.DMA,
        ],
    )
    def grab(table_hbm, idx_hbm, out_hbm, idx_v, buf, sem):
        wid = lax.axis_index("s") * NC + lax.axis_index("c")
        base = wid * b_per_w
        pltpu.sync_copy(idx_hbm.at[pl.ds(wid * rows_per_w, rows_per_w)], idx_v)

        def remap(j, _):
            def remap16(c, _):
                r = idx_v[j, pl.ds(c * 16, 16)]
                p = lax.bitwise_and(r, PC - 1)
                gbase = lax.sub(r, p)
                off = lax.select(
                    p < PH, lax.shift_left(p, 1), 2 * p - (PC - 1)
                )
                idx_v[j, pl.ds(c * 16, 16)] = lax.add(gbase, off)
                return ()

            lax.fori_loop(0, CH // 16, remap16, (), unroll=True)
            return ()

        lax.fori_loop(0, rows_per_w, remap, (), unroll=False)

        def block(g, _):
            for b in range(K):
                pltpu.async_copy(
                    table_hbm.at[idx_v.at[g * K + b]],
                    buf.at[pl.ds(b * CH, CH)],
                    sem,
                )
            for b in range(K):
                pltpu.make_async_copy(
                    table_hbm.at[idx_v.at[g * K + b]],
                    buf.at[pl.ds(b * CH, CH)],
                    sem,
                ).wait()
            pltpu.sync_copy(buf, out_hbm.at[pl.ds(base + g * BLK, BLK)])
            return ()

        lax.fori_loop(0, n_blk, block, (), unroll=False)

    return grab(table, idx2d)


def _project_tc(e2, wbig, Bb, Ll):
    B2, D2 = e2.shape
    KV2 = wbig.shape[1]
    KV = KV2 // 2
    GB = 64
    rows2 = GB * Ll // 2

    def body(x_ref, w_ref, o_ref):
        y = lax.dot_general(
            x_ref[...], w_ref[...],
            dimension_numbers=(((1,), (0,)), ((), ())),
            preferred_element_type=jnp.float32,
        )
        o_ref[...] = y.reshape(rows2, 2, KV).reshape(2 * rows2, KV).reshape(
            GB, Ll, KV
        )

    return pl.pallas_call(
        body,
        grid=(Bb // GB,),
        in_specs=[
            pl.BlockSpec((rows2, D2), lambda i: (i, 0)),
            pl.BlockSpec((D2, KV2), lambda i: (0, 0)),
        ],
        out_specs=pl.BlockSpec((GB, Ll, KV), lambda i: (i, 0, 0)),
        out_shape=jax.ShapeDtypeStruct((Bb, Ll, KV), jnp.float32),
    )(e2, wbig)


def kernel(input_ids, embed_weight, proj_weight):
    Bb, Ll = input_ids.shape
    KV, D = proj_weight.shape
    idx2d = input_ids.reshape(-1, CH).astype(jnp.int32)
    tableP = _pack_table_tc(embed_weight.T)
    table_lin = tableP.reshape(tableP.shape[0] * 2, D)
    e = _gather_sc(table_lin, idx2d)
    e2 = e.reshape(e.shape[0] // 2, 2 * e.shape[1])
    wt = proj_weight.T
    zero = jnp.zeros((D, KV), jnp.float32)
    wbig = jnp.block([[wt, zero], [zero, wt]])
    return _project_tc(e2, wbig, Bb, Ll)


# speedup vs baseline: 1.0766x; 1.0225x over previous
import functools

import jax
import jax.numpy as jnp
from jax import lax
from jax.experimental import pallas as pl
from jax.experimental.pallas import tpu as pltpu
from jax.experimental.pallas import tpu_sc as plsc

NC = 2
NS = 16
NW = NC * NS
CH = 128
K = 4
BLK = CH * K
PC = 32768
PH = PC // 2


def _pack_table_tc(tableT):
    D, V = tableT.shape
    NB = (V + PC - 1) // PC
    def body(x_ref, o_ref):
        x = x_ref[...]
        ya = x[:, :PH].T
        yb = x[:, PH:].T
        o_ref[...] = jnp.concatenate([ya, yb], axis=1)

    return pl.pallas_call(
        body,
        grid=(NB,),
        in_specs=[
            pl.BlockSpec((D, PC), lambda i: (0, i)),
        ],
        out_specs=pl.BlockSpec((PH, 2 * D), lambda i: (i, 0)),
        out_shape=jax.ShapeDtypeStruct((NB * PH, 2 * D), jnp.float32),
    )(tableT)


def _gather_sc(table, idx2d):
    VP, D = table.shape
    B = idx2d.shape[0] * CH
    b_per_w = B // NW
    n_blk = b_per_w // BLK
    rows_per_w = b_per_w // CH

    mesh = plsc.VectorSubcoreMesh(
        core_axis_name="c", subcore_axis_name="s", num_cores=NC, num_subcores=NS
    )

    @functools.partial(
        pl.kernel,
        out_type=jax.ShapeDtypeStruct((B, D), jnp.float32),
        mesh=mesh,
        compiler_params=pltpu.CompilerParams(use_tc_tiling_on_sc=False),
        scratch_types=[
            pltpu.VMEM((rows_per_w, CH), jnp.int32),
            pltpu.VMEM((BLK, D), jnp.float32),
            pltpu.SemaphoreType.DMA,
        ],
    )
    def grab(table_hbm, idx_hbm, out_hbm, idx_v, buf, sem):
        wid = lax.axis_index("s") * NC + lax.axis_index("c")
        base = wid * b_per_w
        pltpu.sync_copy(idx_hbm.at[pl.ds(wid * rows_per_w, rows_per_w)], idx_v)

        def remap(j, _):
            def remap16(c, _):
                r = idx_v[j, pl.ds(c * 16, 16)]
                p = lax.bitwise_and(r, PC - 1)
                gbase = lax.sub(r, p)
                off = lax.select(
                    p < PH, lax.shift_left(p, 1), 2 * p - (PC - 1)
                )
                idx_v[j, pl.ds(c * 16, 16)] = lax.add(gbase, off)
                return ()

            lax.fori_loop(0, CH // 16, remap16, (), unroll=True)
            return ()

        lax.fori_loop(0, rows_per_w, remap, (), unroll=False)

        def block(g, _):
            for b in range(K):
                pltpu.async_copy(
                    table_hbm.at[idx_v.at[g * K + b]],
                    buf.at[pl.ds(b * CH, CH)],
                    sem,
                )
            for b in range(K):
                pltpu.make_async_copy(
                    table_hbm.at[idx_v.at[g * K + b]],
                    buf.at[pl.ds(b * CH, CH)],
                    sem,
                ).wait()
            pltpu.sync_copy(buf, out_hbm.at[pl.ds(base + g * BLK, BLK)])
            return ()

        lax.fori_loop(0, n_blk, block, (), unroll=False)

    return grab(table, idx2d)


def _project_tc(e2, wbig, Bb, Ll):
    B2, D2 = e2.shape
    KV2 = wbig.shape[1]
    KV = KV2 // 2
    GB = 64
    rows2 = GB * Ll // 2

    def body(x_ref, w_ref, o_ref):
        y = lax.dot_general(
            x_ref[...], w_ref[...],
            dimension_numbers=(((1,), (0,)), ((), ())),
            preferred_element_type=jnp.float32,
        )
        o_ref[...] = y.reshape(rows2, 2, KV).reshape(2 * rows2, KV).reshape(
            GB, Ll, KV
        )

    return pl.pallas_call(
        body,
        grid=(Bb // GB,),
        in_specs=[
            pl.BlockSpec((rows2, D2), lambda i: (i, 0)),
            pl.BlockSpec((D2, KV2), lambda i: (0, 0)),
        ],
        out_specs=pl.BlockSpec((GB, Ll, KV), lambda i: (i, 0, 0)),
        out_shape=jax.ShapeDtypeStruct((Bb, Ll, KV), jnp.float32),
    )(e2, wbig)


def kernel(input_ids, embed_weight, proj_weight):
    Bb, Ll = input_ids.shape
    KV, D = proj_weight.shape
    idx2d = input_ids.reshape(-1, CH).astype(jnp.int32)
    tableP = _pack_table_tc(embed_weight.T)
    table_lin = tableP.reshape(tableP.shape[0] * 2, D)
    e = _gather_sc(table_lin, idx2d)
    e2 = e.reshape(e.shape[0] // 2, 2 * e.shape[1])
    wt = proj_weight.T
    zero = jnp.zeros((D, KV), jnp.float32)
    wbig = jnp.block([[wt, zero], [zero, wt]])
    return _project_tc(e2, wbig, Bb, Ll)


# speedup vs baseline: 1.0889x; 1.0114x over previous
import functools

import jax
import jax.numpy as jnp
from jax import lax
from jax.experimental import pallas as pl
from jax.experimental.pallas import tpu as pltpu
from jax.experimental.pallas import tpu_sc as plsc

NC = 2
NS = 16
NW = NC * NS
CH = 128
K = 4
BLK = CH * K
PC = 32768
PH = PC // 2


def _pack_table_tc(tableT):
    D, V = tableT.shape
    NB = (V + PC - 1) // PC
    def body(x_ref, o_ref):
        x = x_ref[...]
        ya = x[:, :PH].T
        yb = x[:, PH:].T
        o_ref[...] = jnp.concatenate([ya, yb], axis=1)

    return pl.pallas_call(
        body,
        grid=(NB,),
        in_specs=[
            pl.BlockSpec((D, PC), lambda i: (0, i)),
        ],
        out_specs=pl.BlockSpec((PH, 2 * D), lambda i: (i, 0)),
        out_shape=jax.ShapeDtypeStruct((NB * PH, 2 * D), jnp.float32),
    )(tableT)


def _gather_sc(table, idx2d):
    VP, D = table.shape
    B = idx2d.shape[0] * CH
    b_per_w = B // NW
    n_blk = b_per_w // BLK
    rows_per_w = b_per_w // CH

    mesh = plsc.VectorSubcoreMesh(
        core_axis_name="c", subcore_axis_name="s", num_cores=NC, num_subcores=NS
    )

    @functools.partial(
        pl.kernel,
        out_type=jax.ShapeDtypeStruct((B, D), jnp.float32),
        mesh=mesh,
        compiler_params=pltpu.CompilerParams(use_tc_tiling_on_sc=False),
        scratch_types=[
            pltpu.VMEM((rows_per_w, CH), jnp.int32),
            pltpu.VMEM((BLK, D), jnp.float32),
            pltpu.SemaphoreType.DMA,
        ],
    )
    def grab(table_hbm, idx_hbm, out_hbm, idx_v, buf, sem):
        wid = lax.axis_index("s") * NC + lax.axis_index("c")
        base = wid * b_per_w
        pltpu.sync_copy(idx_hbm.at[pl.ds(wid * rows_per_w, rows_per_w)], idx_v)

        def remap(j, _):
            def remap16(c, _):
                r = idx_v[j, pl.ds(c * 16, 16)]
                p = lax.bitwise_and(r, PC - 1)
                gbase = lax.sub(r, p)
                off = lax.select(
                    p < PH, lax.shift_left(p, 1), 2 * p - (PC - 1)
                )
                idx_v[j, pl.ds(c * 16, 16)] = lax.add(gbase, off)
                return ()

            lax.fori_loop(0, CH // 16, remap16, (), unroll=True)
            return ()

        lax.fori_loop(0, rows_per_w, remap, (), unroll=False)

        def block(g, _):
            for b in range(K):
                pltpu.async_copy(
                    table_hbm.at[idx_v.at[g * K + b]],
                    buf.at[pl.ds(b * CH, CH)],
                    sem,
                )
            for b in range(K):
                pltpu.make_async_copy(
                    table_hbm.at[idx_v.at[g * K + b]],
                    buf.at[pl.ds(b * CH, CH)],
                    sem,
                ).wait()
            pltpu.sync_copy(buf, out_hbm.at[pl.ds(base + g * BLK, BLK)])
            return ()

        lax.fori_loop(0, n_blk, block, (), unroll=False)

    return grab(table, idx2d)


def _project_tc(e2, wbig, Bb, Ll):
    B2, D2 = e2.shape
    KV2 = wbig.shape[1]
    KV = KV2 // 2
    GB = 128
    rows2 = GB * Ll // 2

    def body(x_ref, w_ref, o_ref):
        y = lax.dot_general(
            x_ref[...], w_ref[...],
            dimension_numbers=(((1,), (0,)), ((), ())),
            preferred_element_type=jnp.float32,
        )
        o_ref[...] = y.reshape(rows2, 2, KV).reshape(2 * rows2, KV).reshape(
            GB, Ll, KV
        )

    return pl.pallas_call(
        body,
        grid=(Bb // GB,),
        in_specs=[
            pl.BlockSpec((rows2, D2), lambda i: (i, 0)),
            pl.BlockSpec((D2, KV2), lambda i: (0, 0)),
        ],
        out_specs=pl.BlockSpec((GB, Ll, KV), lambda i: (i, 0, 0)),
        out_shape=jax.ShapeDtypeStruct((Bb, Ll, KV), jnp.float32),
    )(e2, wbig)


def kernel(input_ids, embed_weight, proj_weight):
    Bb, Ll = input_ids.shape
    KV, D = proj_weight.shape
    idx2d = input_ids.reshape(-1, CH).astype(jnp.int32)
    tableP = _pack_table_tc(embed_weight.T)
    table_lin = tableP.reshape(tableP.shape[0] * 2, D)
    e = _gather_sc(table_lin, idx2d)
    e2 = e.reshape(e.shape[0] // 2, 2 * e.shape[1])
    wt = proj_weight.T
    zero = jnp.zeros((D, KV), jnp.float32)
    wbig = jnp.block([[wt, zero], [zero, wt]])
    return _project_tc(e2, wbig, Bb, Ll)


# speedup vs baseline: 1.0899x; 1.0009x over previous
import functools

import jax
import jax.numpy as jnp
from jax import lax
from jax.experimental import pallas as pl
from jax.experimental.pallas import tpu as pltpu
from jax.experimental.pallas import tpu_sc as plsc

NC = 2
NS = 16
NW = NC * NS
CH = 128
K = 4
BLK = CH * K
PC = 32768
PH = PC // 2


def _pack_table_tc(tableT):
    D, V = tableT.shape
    NB = (V + PC - 1) // PC
    def body(x_ref, o_ref):
        x = x_ref[...]
        ya = x[:, :PH].T
        yb = x[:, PH:].T
        o_ref[...] = jnp.concatenate([ya, yb], axis=1)

    return pl.pallas_call(
        body,
        grid=(NB,),
        in_specs=[
            pl.BlockSpec((D, PC), lambda i: (0, i)),
        ],
        out_specs=pl.BlockSpec((PH, 2 * D), lambda i: (i, 0)),
        out_shape=jax.ShapeDtypeStruct((NB * PH, 2 * D), jnp.float32),
    )(tableT)


def _gather_sc(table, idx2d):
    VP, D = table.shape
    B = idx2d.shape[0] * CH
    b_per_w = B // NW
    n_blk = b_per_w // BLK
    rows_per_w = b_per_w // CH

    mesh = plsc.VectorSubcoreMesh(
        core_axis_name="c", subcore_axis_name="s", num_cores=NC, num_subcores=NS
    )

    @functools.partial(
        pl.kernel,
        out_type=jax.ShapeDtypeStruct((B, D), jnp.float32),
        mesh=mesh,
        compiler_params=pltpu.CompilerParams(use_tc_tiling_on_sc=False),
        scratch_types=[
            pltpu.VMEM((rows_per_w, CH), jnp.int32),
            pltpu.VMEM((BLK, D), jnp.float32),
            pltpu.SemaphoreType.DMA,
        ],
    )
    def grab(table_hbm, idx_hbm, out_hbm, idx_v, buf, sem):
        wid = lax.axis_index("s") * NC + lax.axis_index("c")
        base = wid * b_per_w
        pltpu.sync_copy(idx_hbm.at[pl.ds(wid * rows_per_w, rows_per_w)], idx_v)

        def remap(j, _):
            def remap16(c, _):
                r = idx_v[j, pl.ds(c * 16, 16)]
                p = lax.bitwise_and(r, PC - 1)
                gbase = lax.sub(r, p)
                off = lax.select(
                    p < PH, lax.shift_left(p, 1), 2 * p - (PC - 1)
                )
                idx_v[j, pl.ds(c * 16, 16)] = lax.add(gbase, off)
                return ()

            lax.fori_loop(0, CH // 16, remap16, (), unroll=True)
            return ()

        lax.fori_loop(0, rows_per_w, remap, (), unroll=False)

        def block(g, _):
            for b in range(K):
                pltpu.async_copy(
                    table_hbm.at[idx_v.at[g * K + b]],
                    buf.at[pl.ds(b * CH, CH)],
                    sem,
                )
            for b in range(K):
                pltpu.make_async_copy(
                    table_hbm.at[idx_v.at[g * K + b]],
                    buf.at[pl.ds(b * CH, CH)],
                    sem,
                ).wait()
            pltpu.sync_copy(buf, out_hbm.at[pl.ds(base + g * BLK, BLK)])
            return ()

        lax.fori_loop(0, n_blk, block, (), unroll=False)

    return grab(table, idx2d)


def _project_tc(e2, wbig, Bb, Ll):
    B2, D2 = e2.shape
    KV2 = wbig.shape[1]
    KV = KV2 // 2
    GB = 128
    rows2 = GB * Ll // 2

    def body(x_ref, w_ref, o_ref):
        y = lax.dot_general(
            x_ref[...].astype(jnp.bfloat16), w_ref[...],
            dimension_numbers=(((1,), (0,)), ((), ())),
            preferred_element_type=jnp.float32,
        )
        o_ref[...] = y.reshape(rows2, 2, KV).reshape(2 * rows2, KV).reshape(
            GB, Ll, KV
        )

    return pl.pallas_call(
        body,
        grid=(Bb // GB,),
        in_specs=[
            pl.BlockSpec((rows2, D2), lambda i: (i, 0)),
            pl.BlockSpec((D2, KV2), lambda i: (0, 0)),
        ],
        out_specs=pl.BlockSpec((GB, Ll, KV), lambda i: (i, 0, 0)),
        out_shape=jax.ShapeDtypeStruct((Bb, Ll, KV), jnp.float32),
    )(e2, wbig)


def kernel(input_ids, embed_weight, proj_weight):
    Bb, Ll = input_ids.shape
    KV, D = proj_weight.shape
    idx2d = input_ids.reshape(-1, CH).astype(jnp.int32)
    tableP = _pack_table_tc(embed_weight.T)
    table_lin = tableP.reshape(tableP.shape[0] * 2, D)
    e = _gather_sc(table_lin, idx2d)
    e2 = e.reshape(e.shape[0] // 2, 2 * e.shape[1])
    wt = proj_weight.T
    zero = jnp.zeros((D, KV), jnp.float32)
    wbig = jnp.block([[wt, zero], [zero, wt]]).astype(jnp.bfloat16)
    return _project_tc(e2, wbig, Bb, Ll)


# speedup vs baseline: 1.1419x; 1.0477x over previous
import functools

import jax
import jax.numpy as jnp
from jax import lax
from jax.experimental import pallas as pl
from jax.experimental.pallas import tpu as pltpu
from jax.experimental.pallas import tpu_sc as plsc

NC = 2
NS = 16
NW = NC * NS
CH = 128
K = 4
BLK = CH * K
PC = 32768
PH = PC // 2


def _pack_table_tc(tableT):
    D, V = tableT.shape
    NB = (V + PC - 1) // PC
    def body(x_ref, o_ref):
        x = x_ref[...]
        ya = x[:, :PH].T
        yb = x[:, PH:].T
        o_ref[...] = jnp.concatenate([ya, yb], axis=1)

    return pl.pallas_call(
        body,
        grid=(NB,),
        in_specs=[
            pl.BlockSpec((D, PC), lambda i: (0, i)),
        ],
        out_specs=pl.BlockSpec((PH, 2 * D), lambda i: (i, 0)),
        out_shape=jax.ShapeDtypeStruct((NB * PH, 2 * D), jnp.float32),
    )(tableT)


def _gather_sc(table, idx2d):
    VP, D = table.shape
    B = idx2d.shape[0] * CH
    b_per_w = B // NW
    n_blk = b_per_w // BLK
    rows_per_w = b_per_w // CH

    mesh = plsc.VectorSubcoreMesh(
        core_axis_name="c", subcore_axis_name="s", num_cores=NC, num_subcores=NS
    )

    @functools.partial(
        pl.kernel,
        out_type=jax.ShapeDtypeStruct((B, D), jnp.float32),
        mesh=mesh,
        compiler_params=pltpu.CompilerParams(use_tc_tiling_on_sc=False),
        scratch_types=[
            pltpu.VMEM((rows_per_w, CH), jnp.int32),
            pltpu.VMEM((BLK, D), jnp.float32),
            pltpu.VMEM((BLK, D), jnp.float32),
            pltpu.SemaphoreType.DMA,
            pltpu.SemaphoreType.DMA,
            pltpu.SemaphoreType.DMA,
            pltpu.SemaphoreType.DMA,
        ],
    )
    def grab(table_hbm, idx_hbm, out_hbm, idx_v, buf0, buf1, gs0, gs1, ws0, ws1):
        wid = lax.axis_index("s") * NC + lax.axis_index("c")
        base = wid * b_per_w
        pltpu.sync_copy(idx_hbm.at[pl.ds(wid * rows_per_w, rows_per_w)], idx_v)

        def remap(j, _):
            def remap16(c, _):
                r = idx_v[j, pl.ds(c * 16, 16)]
                p = lax.bitwise_and(r, PC - 1)
                gbase = lax.sub(r, p)
                off = lax.select(
                    p < PH, lax.shift_left(p, 1), 2 * p - (PC - 1)
                )
                idx_v[j, pl.ds(c * 16, 16)] = lax.add(gbase, off)
                return ()

            lax.fori_loop(0, CH // 16, remap16, (), unroll=True)
            return ()

        lax.fori_loop(0, rows_per_w, remap, (), unroll=False)

        bufs = (buf0, buf1)
        gsems = (gs0, gs1)
        wsems = (ws0, ws1)

        def fire(g, p):
            for b in range(K):
                pltpu.async_copy(
                    table_hbm.at[idx_v.at[g * K + b]],
                    bufs[p].at[pl.ds(b * CH, CH)],
                    gsems[p],
                )

        def drain(g, p):
            for b in range(K):
                pltpu.make_async_copy(
                    table_hbm.at[idx_v.at[g * K + b]],
                    bufs[p].at[pl.ds(b * CH, CH)],
                    gsems[p],
                ).wait()

        def put(g, p):
            pltpu.async_copy(
                bufs[p], out_hbm.at[pl.ds(base + g * BLK, BLK)], wsems[p]
            )

        def wait_put(g, p):
            pltpu.make_async_copy(
                bufs[p], out_hbm.at[pl.ds(base + g * BLK, BLK)], wsems[p]
            ).wait()

        fire(0, 0)

        def pair(i, _):
            for p in (0, 1):
                g = 2 * i + p
                drain(g, p)
                put(g, p)
                if p == 0:

                    @pl.when(i >= 1)
                    def _():
                        wait_put(g - 1, 1)

                    fire(g + 1, 1)
                else:

                    @pl.when(i < n_blk // 2 - 1)
                    def _():
                        wait_put(g - 1, 0)
                        fire(g + 1, 0)

            return ()

        lax.fori_loop(0, n_blk // 2, pair, (), unroll=False)
        wait_put(n_blk - 2, 0)
        wait_put(n_blk - 1, 1)

    return grab(table, idx2d)


def _project_tc(e2, wbig, Bb, Ll):
    B2, D2 = e2.shape
    KV2 = wbig.shape[1]
    KV = KV2 // 2
    GB = 128
    rows2 = GB * Ll // 2

    def body(x_ref, w_ref, o_ref):
        y = lax.dot_general(
            x_ref[...], w_ref[...],
            dimension_numbers=(((1,), (0,)), ((), ())),
            preferred_element_type=jnp.float32,
        )
        o_ref[...] = y.reshape(rows2, 2, KV).reshape(2 * rows2, KV).reshape(
            GB, Ll, KV
        )

    return pl.pallas_call(
        body,
        grid=(Bb // GB,),
        in_specs=[
            pl.BlockSpec((rows2, D2), lambda i: (i, 0)),
            pl.BlockSpec((D2, KV2), lambda i: (0, 0)),
        ],
        out_specs=pl.BlockSpec((GB, Ll, KV), lambda i: (i, 0, 0)),
        out_shape=jax.ShapeDtypeStruct((Bb, Ll, KV), jnp.float32),
    )(e2, wbig)


def kernel(input_ids, embed_weight, proj_weight):
    Bb, Ll = input_ids.shape
    KV, D = proj_weight.shape
    idx2d = input_ids.reshape(-1, CH).astype(jnp.int32)
    tableP = _pack_table_tc(embed_weight.T)
    table_lin = tableP.reshape(tableP.shape[0] * 2, D)
    e = _gather_sc(table_lin, idx2d)
    e2 = e.reshape(e.shape[0] // 2, 2 * e.shape[1])
    wt = proj_weight.T
    zero = jnp.zeros((D, KV), jnp.float32)
    wbig = jnp.block([[wt, zero], [zero, wt]])
    return _project_tc(e2, wbig, Bb, Ll)


# speedup vs baseline: 1.2121x; 1.0615x over previous
import functools

import jax
import jax.numpy as jnp
from jax import lax
from jax.experimental import pallas as pl
from jax.experimental.pallas import tpu as pltpu
from jax.experimental.pallas import tpu_sc as plsc

NC = 2
NS = 16
NW = NC * NS
CH = 128
K = 4
BLK = CH * K
PC = 32768
PH = PC // 2


def _pack_table_tc(tableT):
    D, V = tableT.shape
    NB = (V + PC - 1) // PC
    eye = jnp.eye(D, dtype=jnp.bfloat16)

    def body(x_ref, i_ref, o_ref):
        x = x_ref[...].astype(jnp.bfloat16)
        ident = i_ref[...]
        ya = lax.dot_general(
            x[:, :PH], ident, dimension_numbers=(((0,), (0,)), ((), ())),
            preferred_element_type=jnp.float32,
        )
        yb = lax.dot_general(
            x[:, PH:], ident, dimension_numbers=(((0,), (0,)), ((), ())),
            preferred_element_type=jnp.float32,
        )
        o_ref[...] = jnp.concatenate([ya, yb], axis=1)

    return pl.pallas_call(
        body,
        grid=(NB,),
        in_specs=[
            pl.BlockSpec((D, PC), lambda i: (0, i)),
            pl.BlockSpec((D, D), lambda i: (0, 0)),
        ],
        out_specs=pl.BlockSpec((PH, 2 * D), lambda i: (i, 0)),
        out_shape=jax.ShapeDtypeStruct((NB * PH, 2 * D), jnp.float32),
    )(tableT, eye)


def _gather_sc(table, idx2d):
    VP, D = table.shape
    B = idx2d.shape[0] * CH
    b_per_w = B // NW
    n_blk = b_per_w // BLK
    rows_per_w = b_per_w // CH

    mesh = plsc.VectorSubcoreMesh(
        core_axis_name="c", subcore_axis_name="s", num_cores=NC, num_subcores=NS
    )

    @functools.partial(
        pl.kernel,
        out_type=jax.ShapeDtypeStruct((B, D), jnp.float32),
        mesh=mesh,
        compiler_params=pltpu.CompilerParams(use_tc_tiling_on_sc=False),
        scratch_types=[
            pltpu.VMEM((rows_per_w, CH), jnp.int32),
            pltpu.VMEM((BLK, D), jnp.float32),
            pltpu.VMEM((BLK, D), jnp.float32),
            pltpu.SemaphoreType.DMA,
            pltpu.SemaphoreType.DMA,
            pltpu.SemaphoreType.DMA,
            pltpu.SemaphoreType.DMA,
        ],
    )
    def grab(table_hbm, idx_hbm, out_hbm, idx_v, buf0, buf1, gs0, gs1, ws0, ws1):
        wid = lax.axis_index("s") * NC + lax.axis_index("c")
        base = wid * b_per_w
        pltpu.sync_copy(idx_hbm.at[pl.ds(wid * rows_per_w, rows_per_w)], idx_v)

        def remap(j, _):
            def remap16(c, _):
                r = idx_v[j, pl.ds(c * 16, 16)]
                p = lax.bitwise_and(r, PC - 1)
                gbase = lax.sub(r, p)
                off = lax.select(
                    p < PH, lax.shift_left(p, 1), 2 * p - (PC - 1)
                )
                idx_v[j, pl.ds(c * 16, 16)] = lax.add(gbase, off)
                return ()

            lax.fori_loop(0, CH // 16, remap16, (), unroll=True)
            return ()

        lax.fori_loop(0, rows_per_w, remap, (), unroll=False)

        bufs = (buf0, buf1)
        gsems = (gs0, gs1)
        wsems = (ws0, ws1)

        def fire(g, p):
            for b in range(K):
                pltpu.async_copy(
                    table_hbm.at[idx_v.at[g * K + b]],
                    bufs[p].at[pl.ds(b * CH, CH)],
                    gsems[p],
                )

        def drain(g, p):
            for b in range(K):
                pltpu.make_async_copy(
                    table_hbm.at[idx_v.at[g * K + b]],
                    bufs[p].at[pl.ds(b * CH, CH)],
                    gsems[p],
                ).wait()

        def put(g, p):
            pltpu.async_copy(
                bufs[p], out_hbm.at[pl.ds(base + g * BLK, BLK)], wsems[p]
            )

        def wait_put(g, p):
            pltpu.make_async_copy(
                bufs[p], out_hbm.at[pl.ds(base + g * BLK, BLK)], wsems[p]
            ).wait()

        fire(0, 0)

        def pair(i, _):
            for p in (0, 1):
                g = 2 * i + p
                drain(g, p)
                put(g, p)
                if p == 0:

                    @pl.when(i >= 1)
                    def _():
                        wait_put(g - 1, 1)

                    fire(g + 1, 1)
                else:

                    @pl.when(i < n_blk // 2 - 1)
                    def _():
                        wait_put(g - 1, 0)
                        fire(g + 1, 0)

            return ()

        lax.fori_loop(0, n_blk // 2, pair, (), unroll=False)
        wait_put(n_blk - 2, 0)
        wait_put(n_blk - 1, 1)

    return grab(table, idx2d)


def _project_tc(e2, wbig, Bb, Ll):
    B2, D2 = e2.shape
    KV2 = wbig.shape[1]
    KV = KV2 // 2
    GB = 128
    rows2 = GB * Ll // 2

    def body(x_ref, w_ref, o_ref):
        y = lax.dot_general(
            x_ref[...], w_ref[...],
            dimension_numbers=(((1,), (0,)), ((), ())),
            preferred_element_type=jnp.float32,
        )
        o_ref[...] = y.reshape(rows2, 2, KV).reshape(2 * rows2, KV).reshape(
            GB, Ll, KV
        )

    return pl.pallas_call(
        body,
        grid=(Bb // GB,),
        in_specs=[
            pl.BlockSpec((rows2, D2), lambda i: (i, 0)),
            pl.BlockSpec((D2, KV2), lambda i: (0, 0)),
        ],
        out_specs=pl.BlockSpec((GB, Ll, KV), lambda i: (i, 0, 0)),
        out_shape=jax.ShapeDtypeStruct((Bb, Ll, KV), jnp.float32),
    )(e2, wbig)


def kernel(input_ids, embed_weight, proj_weight):
    Bb, Ll = input_ids.shape
    KV, D = proj_weight.shape
    idx2d = input_ids.reshape(-1, CH).astype(jnp.int32)
    tableP = _pack_table_tc(embed_weight.T)
    table_lin = tableP.reshape(tableP.shape[0] * 2, D)
    e = _gather_sc(table_lin, idx2d)
    e2 = e.reshape(e.shape[0] // 2, 2 * e.shape[1])
    wt = proj_weight.T
    zero = jnp.zeros((D, KV), jnp.float32)
    wbig = jnp.block([[wt, zero], [zero, wt]])
    return _project_tc(e2, wbig, Bb, Ll)


# speedup vs baseline: 1.2336x; 1.0178x over previous
import functools

import jax
import jax.numpy as jnp
from jax import lax
from jax.experimental import pallas as pl
from jax.experimental.pallas import tpu as pltpu
from jax.experimental.pallas import tpu_sc as plsc

NC = 2
NS = 16
NW = NC * NS
CH = 128
K = 4
BLK = CH * K
PC = 32768
PH = PC // 2


def _pack_table_tc(tableT):
    D, V = tableT.shape
    NB = (V + PC - 1) // PC
    eye = jnp.eye(D, dtype=jnp.bfloat16)

    def body(x_ref, i_ref, o_ref):
        x = x_ref[...].astype(jnp.bfloat16)
        ident = i_ref[...]
        ya = lax.dot_general(
            x[:, :PH], ident, dimension_numbers=(((0,), (0,)), ((), ())),
            preferred_element_type=jnp.float32,
        )
        yb = lax.dot_general(
            x[:, PH:], ident, dimension_numbers=(((0,), (0,)), ((), ())),
            preferred_element_type=jnp.float32,
        )
        o_ref[...] = jnp.concatenate([ya, yb], axis=1)

    return pl.pallas_call(
        body,
        grid=(NB,),
        in_specs=[
            pl.BlockSpec((D, PC), lambda i: (0, i)),
            pl.BlockSpec((D, D), lambda i: (0, 0)),
        ],
        out_specs=pl.BlockSpec((PH, 2 * D), lambda i: (i, 0)),
        out_shape=jax.ShapeDtypeStruct((NB * PH, 2 * D), jnp.float32),
    )(tableT, eye)


def _gather_sc(table, idx2d):
    VP, D = table.shape
    B = idx2d.shape[0] * CH
    b_per_w = B // NW
    n_blk = b_per_w // BLK
    rows_per_w = b_per_w // CH

    mesh = plsc.VectorSubcoreMesh(
        core_axis_name="c", subcore_axis_name="s", num_cores=NC, num_subcores=NS
    )

    @functools.partial(
        pl.kernel,
        out_type=jax.ShapeDtypeStruct((B, D), jnp.float32),
        mesh=mesh,
        compiler_params=pltpu.CompilerParams(use_tc_tiling_on_sc=False),
        scratch_types=[
            pltpu.VMEM((rows_per_w, CH), jnp.int32),
            pltpu.VMEM((BLK, D), jnp.float32),
            pltpu.VMEM((BLK, D), jnp.float32),
            pltpu.SemaphoreType.DMA,
            pltpu.SemaphoreType.DMA,
            pltpu.SemaphoreType.DMA,
            pltpu.SemaphoreType.DMA,
        ],
    )
    def grab(table_hbm, idx_hbm, out_hbm, idx_v, buf0, buf1, gs0, gs1, ws0, ws1):
        wid = lax.axis_index("s") * NC + lax.axis_index("c")
        base = wid * b_per_w
        pltpu.sync_copy(idx_hbm.at[pl.ds(wid * rows_per_w, rows_per_w)], idx_v)

        def remap(j, _):
            def remap16(c, _):
                r = idx_v[j, pl.ds(c * 16, 16)]
                p = lax.bitwise_and(r, PC - 1)
                gbase = lax.sub(r, p)
                off = lax.select(
                    p < PH, lax.shift_left(p, 1), 2 * p - (PC - 1)
                )
                idx_v[j, pl.ds(c * 16, 16)] = lax.add(gbase, off)
                return ()

            lax.fori_loop(0, CH // 16, remap16, (), unroll=True)
            return ()

        lax.fori_loop(0, rows_per_w, remap, (), unroll=False)

        bufs = (buf0, buf1)
        gsems = (gs0, gs1)
        wsems = (ws0, ws1)

        def fire(g, p):
            for b in range(K):
                pltpu.async_copy(
                    table_hbm.at[idx_v.at[g * K + b]],
                    bufs[p].at[pl.ds(b * CH, CH)],
                    gsems[p],
                )

        def drain(g, p):
            for b in range(K):
                pltpu.make_async_copy(
                    table_hbm.at[idx_v.at[g * K + b]],
                    bufs[p].at[pl.ds(b * CH, CH)],
                    gsems[p],
                ).wait()

        def put(g, p):
            pltpu.async_copy(
                bufs[p], out_hbm.at[pl.ds(base + g * BLK, BLK)], wsems[p]
            )

        def wait_put(g, p):
            pltpu.make_async_copy(
                bufs[p], out_hbm.at[pl.ds(base + g * BLK, BLK)], wsems[p]
            ).wait()

        fire(0, 0)

        def pair(i, _):
            for p in (0, 1):
                g = 2 * i + p
                drain(g, p)
                put(g, p)
                if p == 0:

                    @pl.when(i >= 1)
                    def _():
                        wait_put(g - 1, 1)

                    fire(g + 1, 1)
                else:

                    @pl.when(2 * i + 2 < n_blk)
                    def _():
                        wait_put(g - 1, 0)
                        fire(g + 1, 0)

            return ()

        lax.fori_loop(0, n_blk // 2, pair, (), unroll=False)
        if n_blk % 2:
            gt = n_blk - 1
            drain(gt, 0)
            put(gt, 0)
            wait_put(gt - 1, 1)
            wait_put(gt, 0)
        else:
            wait_put(n_blk - 2, 0)
            wait_put(n_blk - 1, 1)

    return grab(table, idx2d)


GB = 128


def _project_tc_first(e2, wbig, Bb, Ll):
    B2, D2 = e2.shape
    KV2 = wbig.shape[1]
    KV = KV2 // 2
    rows2 = GB * Ll // 2

    def body(x_ref, w_ref, o_ref):
        y = lax.dot_general(
            x_ref[...], w_ref[...],
            dimension_numbers=(((1,), (0,)), ((), ())),
            preferred_element_type=jnp.float32,
        )
        o_ref[...] = y.reshape(rows2, 2, KV).reshape(2 * rows2, KV).reshape(
            GB, Ll, KV
        )

    return pl.pallas_call(
        body,
        grid=(B2 // rows2,),
        in_specs=[
            pl.BlockSpec((rows2, D2), lambda i: (i, 0)),
            pl.BlockSpec((D2, KV2), lambda i: (0, 0)),
        ],
        out_specs=pl.BlockSpec((GB, Ll, KV), lambda i: (i, 0, 0)),
        out_shape=jax.ShapeDtypeStruct((Bb, Ll, KV), jnp.float32),
    )(e2, wbig)


def _project_tc_second(out1, e2, wbig, Bb, Ll):
    B2, D2 = e2.shape
    KV2 = wbig.shape[1]
    KV = KV2 // 2
    rows2 = GB * Ll // 2
    half = Bb // (2 * GB)

    def body(prev_ref, x_ref, w_ref, o_ref):
        del prev_ref
        y = lax.dot_general(
            x_ref[...], w_ref[...],
            dimension_numbers=(((1,), (0,)), ((), ())),
            preferred_element_type=jnp.float32,
        )
        o_ref[...] = y.reshape(rows2, 2, KV).reshape(2 * rows2, KV).reshape(
            GB, Ll, KV
        )

    return pl.pallas_call(
        body,
        grid=(B2 // rows2,),
        in_specs=[
            pl.BlockSpec(memory_space=pl.ANY),
            pl.BlockSpec((rows2, D2), lambda i: (i, 0)),
            pl.BlockSpec((D2, KV2), lambda i: (0, 0)),
        ],
        out_specs=pl.BlockSpec((GB, Ll, KV), lambda i: (i + half, 0, 0)),
        out_shape=jax.ShapeDtypeStruct((Bb, Ll, KV), jnp.float32),
        input_output_aliases={0: 0},
    )(out1, e2, wbig)


def kernel(input_ids, embed_weight, proj_weight):
    Bb, Ll = input_ids.shape
    KV, D = proj_weight.shape
    idx2d = input_ids.reshape(-1, CH).astype(jnp.int32)
    half_rows = idx2d.shape[0] // 2
    tableP = _pack_table_tc(embed_weight.T)
    table_lin = tableP.reshape(tableP.shape[0] * 2, D)
    e_a = _gather_sc(table_lin, idx2d[:half_rows])
    e_b = _gather_sc(table_lin, idx2d[half_rows:])
    e2_a = e_a.reshape(e_a.shape[0] // 2, 2 * D)
    e2_b = e_b.reshape(e_b.shape[0] // 2, 2 * D)
    wt = proj_weight.T
    zero = jnp.zeros((D, KV), jnp.float32)
    wbig = jnp.block([[wt, zero], [zero, wt]])
    out1 = _project_tc_first(e2_a, wbig, Bb, Ll)
    return _project_tc_second(out1, e2_b, wbig, Bb, Ll)
